# Initial kernel scaffold; baseline (speedup 1.0000x reference)
import jax
import jax.numpy as jnp
from jax.experimental import pallas as pl


def kernel(h, edge_index, W1, bn1g, bn1b, W2, bng, bnb, Wp, bp):
    raise NotImplementedError("write your pallas kernel here")


# speedup vs baseline: 9.0617x; 9.0617x over previous
import functools

import jax
import jax.numpy as jnp
from jax import lax
from jax.experimental import pallas as pl
from jax.experimental.pallas import tpu as pltpu
from jax.experimental.pallas import tpu_sc as plsc

N = 10000
E = 320000
D = 128
L = 4

NC = 2
NS = 16
NW = NC * NS
EPW = E // NW
CH = 80
NCHUNK = EPW // CH
RPT = 624


SLABC = 25
SLAB_E = SLABC * CH
NSLAB = NCHUNK // SLABC


def _sc_aggregate(x, src, dst, zeros):
    mesh = plsc.VectorSubcoreMesh(core_axis_name="c", subcore_axis_name="s")

    @functools.partial(
        pl.kernel,
        out_type=jax.ShapeDtypeStruct((NC, N, D), jnp.float32),
        mesh=mesh,
        scratch_types=[
            pltpu.VMEM((SLAB_E,), jnp.int32),
            pltpu.VMEM((SLAB_E,), jnp.int32),
            pltpu.VMEM((SLAB_E,), jnp.int32),
            pltpu.VMEM((SLAB_E,), jnp.int32),
            pltpu.VMEM((CH,), jnp.int32),
            pltpu.VMEM((CH,), jnp.int32),
            pltpu.VMEM((CH, D), jnp.float32),
            pltpu.VMEM((CH, D), jnp.float32),
            pltpu.VMEM_SHARED((N, D), jnp.float32),
            pltpu.SemaphoreType.DMA,
            pltpu.SemaphoreType.DMA,
            pltpu.SemaphoreType.DMA,
        ],
    )
    def k(x_hbm, src_hbm, dst_hbm, z_hbm, out_hbm,
          srcs0, srcs1, dsts0, dsts1, dstb0, dstb1, rows0, rows1,
          agg_sh, sem_slab, semr0, semr1):
        cid = lax.axis_index("c")
        sid = lax.axis_index("s")
        wid = sid * NC + cid
        ebase = pl.multiple_of(wid * EPW, 8)

        r0 = pl.multiple_of(sid * RPT, 8)
        pltpu.sync_copy(z_hbm.at[pl.ds(r0, RPT)], agg_sh.at[pl.ds(r0, RPT)])

        @pl.when(sid == NS - 1)
        def _zero_tail():
            pltpu.sync_copy(z_hbm.at[pl.ds(NS * RPT, N - NS * RPT)],
                            agg_sh.at[pl.ds(NS * RPT, N - NS * RPT)])

        srcs = (srcs0, srcs1)
        dsts = (dsts0, dsts1)

        def load_slab(s, b, copy):
            off = ebase + s * SLAB_E
            copy(src_hbm.at[pl.ds(off, SLAB_E)], srcs[b])
            copy(dst_hbm.at[pl.ds(off, SLAB_E)], dsts[b])

        def drain_slab(b):
            pltpu.make_async_copy(src_hbm.at[pl.ds(0, SLAB_E)], srcs[b],
                                  sem_slab).wait()
            pltpu.make_async_copy(dst_hbm.at[pl.ds(0, SLAB_E)], dsts[b],
                                  sem_slab).wait()

        load_slab(0, 0, pltpu.sync_copy)
        plsc.subcore_barrier()

        def fire(sb, j, rows_buf, sem):
            pltpu.async_copy(x_hbm.at[srcs[sb].at[pl.ds(j * CH, CH)]],
                             rows_buf, sem)

        def drain_rows(rows_buf, sem):
            pltpu.make_async_copy(x_hbm.at[pl.ds(0, CH)], rows_buf, sem).wait()

        def scatter(sb, j, rows_buf, dstb):
            for g in range(CH // 16):
                dstb[pl.ds(g * 16, 16)] = dsts[sb][pl.ds(j * CH + g * 16, 16)]
            pltpu.sync_copy(rows_buf, agg_sh.at[dstb], add=True)

        for s in range(NSLAB):
            sb = s % 2
            if (s * SLABC) % 2 == 0:
                rA, rB, dA, dB, sA, sB = rows0, rows1, dstb0, dstb1, semr0, semr1
            else:
                rA, rB, dA, dB, sA, sB = rows1, rows0, dstb1, dstb0, semr1, semr0
            if s + 1 < NSLAB:
                load_slab(s + 1, (s + 1) % 2,
                          lambda a, b: pltpu.async_copy(a, b, sem_slab))

            fire(sb, 0, rA, sA)

            def pair(p, carry, sb=sb, rA=rA, rB=rB, dA=dA, dB=dB, sA=sA, sB=sB):
                c0 = 2 * p
                fire(sb, c0 + 1, rB, sB)
                drain_rows(rA, sA)
                scatter(sb, c0, rA, dA)
                fire(sb, c0 + 2, rA, sA)
                drain_rows(rB, sB)
                scatter(sb, c0 + 1, rB, dB)
                return carry

            lax.fori_loop(0, (SLABC - 1) // 2, pair, 0)
            drain_rows(rA, sA)
            scatter(sb, SLABC - 1, rA, dA)

            if s + 1 < NSLAB:
                drain_slab((s + 1) % 2)

        plsc.subcore_barrier()
        pltpu.sync_copy(agg_sh.at[pl.ds(r0, RPT)],
                        out_hbm.at[cid, pl.ds(r0, RPT)])

        @pl.when(sid == NS - 1)
        def _out_tail():
            pltpu.sync_copy(agg_sh.at[pl.ds(NS * RPT, N - NS * RPT)],
                            out_hbm.at[cid, pl.ds(NS * RPT, N - NS * RPT)])

    return k(x, src, dst, zeros)


def _tc_layer(x, parts, W1, g1, b1, W2, g2, b2):
    def body(x_ref, p_ref, w1_ref, g1_ref, b1_ref, w2_ref, g2_ref, b2_ref,
             out_ref, pool_ref):
        y = x_ref[...] + p_ref[0] + p_ref[1]
        t = jnp.dot(y, w1_ref[...], preferred_element_type=jnp.float32, precision=lax.Precision.HIGHEST)
        m = jnp.mean(t, axis=0, keepdims=True)
        v = jnp.mean((t - m) ** 2, axis=0, keepdims=True)
        t = g1_ref[...] * (t - m) * lax.rsqrt(v + 1e-5) + b1_ref[...]
        t = jnp.maximum(t, 0.0)
        u = jnp.dot(t, w2_ref[...], preferred_element_type=jnp.float32, precision=lax.Precision.HIGHEST)
        m2 = jnp.mean(u, axis=0, keepdims=True)
        v2 = jnp.mean((u - m2) ** 2, axis=0, keepdims=True)
        u = g2_ref[...] * (u - m2) * lax.rsqrt(v2 + 1e-5) + b2_ref[...]
        u = jnp.maximum(u, 0.0)
        out_ref[...] = u
        pool_ref[...] = jnp.max(u, axis=0, keepdims=True)

    return pl.pallas_call(
        body,
        out_shape=(jax.ShapeDtypeStruct((N, D), jnp.float32),
                   jax.ShapeDtypeStruct((1, D), jnp.float32)),
    )(x, parts, W1, g1, b1, W2, g2, b2)


def _tc_readout(h, pools, Wp, bp):
    def body(h_ref, pools_ref, wp_ref, bp_ref, out_ref):
        p0 = jnp.max(h_ref[...], axis=0, keepdims=True)
        acc = jnp.dot(p0, wp_ref[0], preferred_element_type=jnp.float32, precision=lax.Precision.HIGHEST)
        acc = acc + bp_ref[pl.ds(0, 1), :]
        for i in range(L):
            pi = pools_ref[pl.ds(i, 1), :]
            acc = acc + jnp.dot(pi, wp_ref[i + 1],
                                preferred_element_type=jnp.float32, precision=lax.Precision.HIGHEST)
            acc = acc + bp_ref[pl.ds(i + 1, 1), :]
        out_ref[...] = acc

    return pl.pallas_call(
        body,
        out_shape=jax.ShapeDtypeStruct((1, D), jnp.float32),
    )(h, pools, Wp, bp)


def kernel(h, edge_index, W1, bn1g, bn1b, W2, bng, bnb, Wp, bp):
    src = edge_index[0]
    dst = edge_index[1]
    zeros = jnp.zeros((N, D), dtype=jnp.float32)

    x = h
    pools = []
    for i in range(L):
        parts = _sc_aggregate(x, src, dst, zeros)
        x, pool = _tc_layer(x, parts,
                            W1[i], bn1g[i].reshape(1, D), bn1b[i].reshape(1, D),
                            W2[i], bng[i].reshape(1, D), bnb[i].reshape(1, D))
        pools.append(pool)

    pools = jnp.concatenate(pools, axis=0)
    return _tc_readout(h, pools, Wp, bp)


# speedup vs baseline: 9.4822x; 1.0464x over previous
import functools

import jax
import jax.numpy as jnp
from jax import lax
from jax.experimental import pallas as pl
from jax.experimental.pallas import tpu as pltpu
from jax.experimental.pallas import tpu_sc as plsc

N = 10000
E = 320000
D = 128
L = 4

NC = 2
NS = 16
NW = NC * NS
EPW = E // NW
CH = 40
NB = 5
NG = EPW // (CH * NB)
RPT = 624


def _sc_aggregate(x, src, dst, zeros):
    mesh = plsc.VectorSubcoreMesh(core_axis_name="c", subcore_axis_name="s")

    @functools.partial(
        pl.kernel,
        out_type=jax.ShapeDtypeStruct((NC, N, D), jnp.float32),
        mesh=mesh,
        scratch_types=(
            [pltpu.VMEM((CH,), jnp.int32) for _ in range(NB)]
            + [pltpu.VMEM((CH,), jnp.int32) for _ in range(NB)]
            + [pltpu.VMEM((CH, D), jnp.float32) for _ in range(NB)]
            + [pltpu.VMEM_SHARED((N, D), jnp.float32)]
            + [pltpu.SemaphoreType.DMA for _ in range(3 * NB)]
        ),
    )
    def k(x_hbm, src_hbm, dst_hbm, z_hbm, out_hbm, *bufs):
        srcb = bufs[0:NB]
        dstb = bufs[NB:2 * NB]
        rows = bufs[2 * NB:3 * NB]
        agg_sh = bufs[3 * NB]
        isem = bufs[3 * NB + 1:3 * NB + 1 + NB]
        gsem = bufs[3 * NB + 1 + NB:3 * NB + 1 + 2 * NB]
        ssem = bufs[3 * NB + 1 + 2 * NB:3 * NB + 1 + 3 * NB]

        cid = lax.axis_index("c")
        sid = lax.axis_index("s")
        wid = sid * NC + cid
        ebase = pl.multiple_of(wid * EPW, 8)

        def fire_idx(j, b):
            off = ebase + j * CH
            pltpu.async_copy(src_hbm.at[pl.ds(off, CH)], srcb[b], isem[b])
            pltpu.async_copy(dst_hbm.at[pl.ds(off, CH)], dstb[b], isem[b])

        def wait_idx(b):
            pltpu.make_async_copy(src_hbm.at[pl.ds(0, CH)], srcb[b],
                                  isem[b]).wait()
            pltpu.make_async_copy(dst_hbm.at[pl.ds(0, CH)], dstb[b],
                                  isem[b]).wait()

        def fire_gather(b):
            pltpu.async_copy(x_hbm.at[srcb[b]], rows[b], gsem[b])

        def wait_gather(b):
            pltpu.make_async_copy(x_hbm.at[pl.ds(0, CH)], rows[b],
                                  gsem[b]).wait()

        def fire_scatter(b):
            pltpu.async_copy(rows[b], agg_sh.at[dstb[b]], ssem[b], add=True)

        def wait_scatter(b):
            pltpu.make_async_copy(x_hbm.at[pl.ds(0, CH)], rows[b],
                                  ssem[b]).wait()

        r0 = pl.multiple_of(sid * RPT, 8)
        pltpu.sync_copy(z_hbm.at[pl.ds(r0, RPT)], agg_sh.at[pl.ds(r0, RPT)])

        @pl.when(sid == NS - 1)
        def _zero_tail():
            pltpu.sync_copy(z_hbm.at[pl.ds(NS * RPT, N - NS * RPT)],
                            agg_sh.at[pl.ds(NS * RPT, N - NS * RPT)])

        for b in range(NB):
            fire_idx(b, b)

        plsc.subcore_barrier()

        for b in range(NB):
            wait_idx(b)
            fire_gather(b)

        def group(g, carry):
            for b in range(NB):
                wait_gather(b)
                fire_scatter(b)

                @pl.when(g + 1 < NG)
                def _prefetch(g=g, b=b):
                    fire_idx((g + 1) * NB + b, b)

            for b in range(NB):
                @pl.when(g + 1 < NG)
                def _refill(g=g, b=b):
                    wait_scatter(b)
                    wait_idx(b)
                    fire_gather(b)

            return carry

        lax.fori_loop(0, NG, group, 0)

        for b in range(NB):
            wait_scatter(b)

        plsc.subcore_barrier()
        pltpu.sync_copy(agg_sh.at[pl.ds(r0, RPT)],
                        out_hbm.at[cid, pl.ds(r0, RPT)])

        @pl.when(sid == NS - 1)
        def _out_tail():
            pltpu.sync_copy(agg_sh.at[pl.ds(NS * RPT, N - NS * RPT)],
                            out_hbm.at[cid, pl.ds(NS * RPT, N - NS * RPT)])

    return k(x, src, dst, zeros)


def _tc_layer(x, parts, W1, g1, b1, W2, g2, b2):
    def body(x_ref, p_ref, w1_ref, g1_ref, b1_ref, w2_ref, g2_ref, b2_ref,
             out_ref, pool_ref):
        y = x_ref[...] + p_ref[0] + p_ref[1]
        t = jnp.dot(y, w1_ref[...], preferred_element_type=jnp.float32, precision=lax.Precision.HIGHEST)
        m = jnp.mean(t, axis=0, keepdims=True)
        v = jnp.mean((t - m) ** 2, axis=0, keepdims=True)
        t = g1_ref[...] * (t - m) * lax.rsqrt(v + 1e-5) + b1_ref[...]
        t = jnp.maximum(t, 0.0)
        u = jnp.dot(t, w2_ref[...], preferred_element_type=jnp.float32, precision=lax.Precision.HIGHEST)
        m2 = jnp.mean(u, axis=0, keepdims=True)
        v2 = jnp.mean((u - m2) ** 2, axis=0, keepdims=True)
        u = g2_ref[...] * (u - m2) * lax.rsqrt(v2 + 1e-5) + b2_ref[...]
        u = jnp.maximum(u, 0.0)
        out_ref[...] = u
        pool_ref[...] = jnp.max(u, axis=0, keepdims=True)

    return pl.pallas_call(
        body,
        out_shape=(jax.ShapeDtypeStruct((N, D), jnp.float32),
                   jax.ShapeDtypeStruct((1, D), jnp.float32)),
    )(x, parts, W1, g1, b1, W2, g2, b2)


def _tc_readout(h, pools, Wp, bp):
    def body(h_ref, pools_ref, wp_ref, bp_ref, out_ref):
        p0 = jnp.max(h_ref[...], axis=0, keepdims=True)
        acc = jnp.dot(p0, wp_ref[0], preferred_element_type=jnp.float32, precision=lax.Precision.HIGHEST)
        acc = acc + bp_ref[pl.ds(0, 1), :]
        for i in range(L):
            pi = pools_ref[pl.ds(i, 1), :]
            acc = acc + jnp.dot(pi, wp_ref[i + 1],
                                preferred_element_type=jnp.float32, precision=lax.Precision.HIGHEST)
            acc = acc + bp_ref[pl.ds(i + 1, 1), :]
        out_ref[...] = acc

    return pl.pallas_call(
        body,
        out_shape=jax.ShapeDtypeStruct((1, D), jnp.float32),
    )(h, pools, Wp, bp)


def kernel(h, edge_index, W1, bn1g, bn1b, W2, bng, bnb, Wp, bp):
    src = edge_index[0]
    dst = edge_index[1]
    zeros = jnp.zeros((N, D), dtype=jnp.float32)

    x = h
    pools = []
    for i in range(L):
        parts = _sc_aggregate(x, src, dst, zeros)
        x, pool = _tc_layer(x, parts,
                            W1[i], bn1g[i].reshape(1, D), bn1b[i].reshape(1, D),
                            W2[i], bng[i].reshape(1, D), bnb[i].reshape(1, D))
        pools.append(pool)

    pools = jnp.concatenate(pools, axis=0)
    return _tc_readout(h, pools, Wp, bp)
